# Initial kernel scaffold; baseline (speedup 1.0000x reference)
import jax
import jax.numpy as jnp
from jax.experimental import pallas as pl


def kernel(element_tokens, count_tokens, element_table, count_table):
    raise NotImplementedError("write your pallas kernel here")


# speedup vs baseline: 3.7498x; 3.7498x over previous
import functools

import jax
import jax.numpy as jnp
from jax import lax
from jax.experimental import pallas as pl
from jax.experimental.pallas import tpu as pltpu
from jax.experimental.pallas import tpu_sc as plsc

DIM = 512
N_ELEMENTS = 118
N_ELEM_PAD = 128
N_COUNT = 17
N_PAIR = N_ELEM_PAD * N_COUNT
BATCH = 4096
FORMULA_LEN = 16
TOKENS = BATCH * FORMULA_LEN
NUM_CORES = 2
NUM_SUBCORES = 16
NW = NUM_CORES * NUM_SUBCORES
TOK_PER_W = TOKENS // NW
CHUNK = 64
N_CHUNK = TOK_PER_W // CHUNK


def _pair_table_tc(elem_pad, count_table):
    def body(e_ref, c_ref, o_ref):
        o_ref[...] = e_ref[...][:, None, :] + c_ref[...][None, :, :]

    return pl.pallas_call(
        body,
        out_shape=jax.ShapeDtypeStruct((N_ELEM_PAD, N_COUNT, DIM), jnp.float32),
    )(elem_pad, count_table)


_SC_MESH = plsc.VectorSubcoreMesh(core_axis_name="c", subcore_axis_name="s")


@functools.partial(
    pl.kernel,
    out_type=jax.ShapeDtypeStruct((TOKENS, DIM), jnp.float32),
    mesh=_SC_MESH,
    scratch_types=[
        pltpu.VMEM((CHUNK,), jnp.int32),
        pltpu.VMEM((CHUNK,), jnp.int32),
        pltpu.VMEM((CHUNK,), jnp.int32),
        pltpu.VMEM((CHUNK, DIM), jnp.float32),
        pltpu.SemaphoreType.DMA,
    ],
)
def _sc_gather(pair_hbm, etok_hbm, ctok_hbm, out_hbm,
               ebuf, cbuf, pbuf, obuf, sem):
    cid = lax.axis_index("c")
    sid = lax.axis_index("s")
    wid = cid * NUM_SUBCORES + sid

    base = wid * TOK_PER_W

    def chunk_body(k, carry):
        off = base + k * CHUNK
        pltpu.sync_copy(etok_hbm.at[pl.ds(off, CHUNK)], ebuf)
        pltpu.sync_copy(ctok_hbm.at[pl.ds(off, CHUNK)], cbuf)
        for j in range(CHUNK // 16):
            sl = pl.ds(j * 16, 16)
            pbuf[sl] = ebuf[sl] * N_COUNT + cbuf[sl]
        pltpu.async_copy(pair_hbm.at[pbuf], obuf, sem).wait()
        pltpu.sync_copy(obuf, out_hbm.at[pl.ds(off, CHUNK)])
        return carry

    lax.fori_loop(0, N_CHUNK, chunk_body, 0)


def kernel(element_tokens, count_tokens, element_table, count_table):
    elem_pad = jnp.zeros((N_ELEM_PAD, DIM), jnp.float32).at[:N_ELEMENTS].set(
        element_table)
    pair = _pair_table_tc(elem_pad, count_table).reshape(N_PAIR, DIM)
    et = element_tokens.reshape(-1).astype(jnp.int32)
    ct = count_tokens.reshape(-1).astype(jnp.int32)
    out = _sc_gather(pair, et, ct)
    return out.reshape(BATCH, FORMULA_LEN, DIM)
